# baseline (speedup 1.0000x reference)
import functools

import jax
import jax.numpy as jnp
from jax import lax
from jax.experimental import pallas as pl
from jax.experimental.pallas import tpu as pltpu
from jax.experimental.pallas import tpu_sc as plsc

N_NODE = 10000
N_PAD = 10240
N_EDGE = 320000
HID = 128

NC = 2
NS = 16
NW = NC * NS
EPT = N_EDGE // NW
CH = 80
NCHUNK = EPT // CH
RPT = N_PAD // NS

@functools.cache
def _mesh():
    return plsc.VectorSubcoreMesh(
        core_axis_name="c", subcore_axis_name="s", num_cores=NC, num_subcores=NS
    )

BN = 1024
BE = 2000


def _zero_rows(zb_v, nrows, ncolgrp):
    @pl.loop(0, nrows)
    def _(r):
        @pl.loop(0, ncolgrp)
        def _(j):
            zb_v[r, pl.ds(j * 16, 16)] = jnp.zeros((16,), jnp.float32)


def _count_body(dst_hbm, ones_hbm, out_hbm, idx_v, ones_v, zb_v, acc):
    cid = lax.axis_index("c")
    sid = lax.axis_index("s")
    wid = cid * NS + sid

    pltpu.sync_copy(ones_hbm, ones_v)
    _zero_rows(zb_v, 64, 8)

    @pl.loop(0, RPT // 64)
    def _(k):
        pltpu.sync_copy(zb_v, acc.at[pl.ds(sid * RPT + k * 64, 64)])

    plsc.subcore_barrier()
    ebase = wid * EPT

    @pl.loop(0, NCHUNK)
    def _(c):
        pltpu.sync_copy(dst_hbm.at[pl.ds(ebase + c * CH, CH)], idx_v)
        pltpu.sync_copy(ones_v, acc.at[idx_v], add=True)

    plsc.subcore_barrier()
    pltpu.sync_copy(acc.at[pl.ds(sid * RPT, RPT)],
                    out_hbm.at[cid, pl.ds(sid * RPT, RPT)])


def _sc_count(dst):
    f = pl.kernel(
        _count_body,
        out_type=jax.ShapeDtypeStruct((NC, N_PAD, HID), jnp.float32),
        mesh=_mesh(),
        scratch_types=[
            pltpu.VMEM((CH,), jnp.int32),
            pltpu.VMEM((CH, HID), jnp.float32),
            pltpu.VMEM((64, HID), jnp.float32),
            pltpu.VMEM_SHARED((N_PAD, HID), jnp.float32),
        ],
    )
    return f(dst, jnp.ones((CH, HID), jnp.float32))


def _gather_body(idx_hbm, tab_hbm, out_hbm, idx_v, rows_v):
    wid = lax.axis_index("c") * NS + lax.axis_index("s")
    ebase = wid * EPT

    @pl.loop(0, NCHUNK)
    def _(c):
        eb = ebase + c * CH
        pltpu.sync_copy(idx_hbm.at[pl.ds(eb, CH)], idx_v)
        pltpu.sync_copy(tab_hbm.at[idx_v], rows_v)
        pltpu.sync_copy(rows_v, out_hbm.at[pl.ds(eb, CH)])


def _sc_gather(idx, tab):
    width = tab.shape[1]
    f = pl.kernel(
        _gather_body,
        out_type=jax.ShapeDtypeStruct((N_EDGE, width), tab.dtype),
        mesh=_mesh(),
        scratch_types=[
            pltpu.VMEM((CH,), jnp.int32),
            pltpu.VMEM((CH, width), tab.dtype),
        ],
    )
    return f(idx, tab)


def _gather_scatter_body(srcidx_hbm, dstidx_hbm, tab_hbm, out_hbm,
                         isv, idv, rows_v, zb_v, acc):
    cid = lax.axis_index("c")
    sid = lax.axis_index("s")
    wid = cid * NS + sid

    _zero_rows(zb_v, 64, 8)

    @pl.loop(0, RPT // 64)
    def _(k):
        pltpu.sync_copy(zb_v, acc.at[pl.ds(sid * RPT + k * 64, 64)])

    plsc.subcore_barrier()
    ebase = wid * EPT

    @pl.loop(0, NCHUNK)
    def _(c):
        eb = ebase + c * CH
        pltpu.sync_copy(srcidx_hbm.at[pl.ds(eb, CH)], isv)
        pltpu.sync_copy(dstidx_hbm.at[pl.ds(eb, CH)], idv)
        pltpu.sync_copy(tab_hbm.at[isv], rows_v)
        pltpu.sync_copy(rows_v, acc.at[idv], add=True)

    plsc.subcore_barrier()
    pltpu.sync_copy(acc.at[pl.ds(sid * RPT, RPT)],
                    out_hbm.at[cid, pl.ds(sid * RPT, RPT)])


def _sc_gather_scatter(src, dst, tab):
    f = pl.kernel(
        _gather_scatter_body,
        out_type=jax.ShapeDtypeStruct((NC, N_PAD, HID), jnp.float32),
        mesh=_mesh(),
        scratch_types=[
            pltpu.VMEM((CH,), jnp.int32),
            pltpu.VMEM((CH,), jnp.int32),
            pltpu.VMEM((CH, HID), jnp.float32),
            pltpu.VMEM((64, HID), jnp.float32),
            pltpu.VMEM_SHARED((N_PAD, HID), jnp.float32),
        ],
    )
    return f(src, dst, tab)


def _scatter_body(idx_hbm, val_hbm, out_hbm, idx_v, val_v, zb_v, acc):
    cid = lax.axis_index("c")
    sid = lax.axis_index("s")
    wid = cid * NS + sid

    _zero_rows(zb_v, 64, 8)

    @pl.loop(0, RPT // 64)
    def _(k):
        pltpu.sync_copy(zb_v, acc.at[pl.ds(sid * RPT + k * 64, 64)])

    plsc.subcore_barrier()
    ebase = wid * EPT

    @pl.loop(0, NCHUNK)
    def _(c):
        eb = ebase + c * CH
        pltpu.sync_copy(idx_hbm.at[pl.ds(eb, CH)], idx_v)
        pltpu.sync_copy(val_hbm.at[pl.ds(eb, CH)], val_v)
        pltpu.sync_copy(val_v, acc.at[idx_v], add=True)

    plsc.subcore_barrier()
    pltpu.sync_copy(acc.at[pl.ds(sid * RPT, RPT)],
                    out_hbm.at[cid, pl.ds(sid * RPT, RPT)])


def _sc_scatter(idx, vals):
    f = pl.kernel(
        _scatter_body,
        out_type=jax.ShapeDtypeStruct((NC, N_PAD, HID), jnp.float32),
        mesh=_mesh(),
        scratch_types=[
            pltpu.VMEM((CH,), jnp.int32),
            pltpu.VMEM((CH, HID), jnp.float32),
            pltpu.VMEM((64, HID), jnp.float32),
            pltpu.VMEM_SHARED((N_PAD, HID), jnp.float32),
        ],
    )
    return f(idx, vals)


def _deg_body(p_ref, dinv_ref, recip_ref):
    p = p_ref[...]
    cnt = (p[0] + p[1])[:, 0:1]
    dinv = lax.rsqrt(cnt + 1.0)
    recip = 1.0 / jnp.maximum(cnt, 1.0)
    dinv_ref[...] = jnp.broadcast_to(dinv, dinv_ref.shape)
    recip_ref[...] = jnp.broadcast_to(recip, recip_ref.shape)


def _tc_deg(parts):
    grid = (N_PAD // BN,)
    out = jax.ShapeDtypeStruct((N_PAD, HID), jnp.float32)
    return pl.pallas_call(
        _deg_body,
        grid=grid,
        in_specs=[pl.BlockSpec((NC, BN, HID), lambda i: (0, i, 0))],
        out_specs=[pl.BlockSpec((BN, HID), lambda i: (i, 0))] * 2,
        out_shape=[out, out],
    )(parts)


def _mm_scale_body(x_ref, w_ref, s_ref, o_ref):
    xw = jnp.dot(x_ref[...], w_ref[...], preferred_element_type=jnp.float32)
    o_ref[...] = xw * s_ref[...]


def _tc_mm_scale(x, w, s):
    grid = (N_PAD // BN,)
    return pl.pallas_call(
        _mm_scale_body,
        grid=grid,
        in_specs=[
            pl.BlockSpec((BN, x.shape[1]), lambda i: (i, 0)),
            pl.BlockSpec(w.shape, lambda i: (0, 0)),
            pl.BlockSpec((BN, HID), lambda i: (i, 0)),
        ],
        out_specs=pl.BlockSpec((BN, w.shape[1]), lambda i: (i, 0)),
        out_shape=jax.ShapeDtypeStruct((N_PAD, w.shape[1]), jnp.float32),
    )(x, w, s)


def _gcn_comb_body(p_ref, xs_ref, dinv_ref, b_ref, o_ref):
    p = p_ref[...]
    s = p[0] + p[1] + xs_ref[...]
    o_ref[...] = jnp.maximum(dinv_ref[...] * s + b_ref[...], 0.0)


def _tc_gcn_combine(parts, xs, dinv_b, bias):
    grid = (N_PAD // BN,)
    return pl.pallas_call(
        _gcn_comb_body,
        grid=grid,
        in_specs=[
            pl.BlockSpec((NC, BN, HID), lambda i: (0, i, 0)),
            pl.BlockSpec((BN, HID), lambda i: (i, 0)),
            pl.BlockSpec((BN, HID), lambda i: (i, 0)),
            pl.BlockSpec((1, HID), lambda i: (0, 0)),
        ],
        out_specs=pl.BlockSpec((BN, HID), lambda i: (i, 0)),
        out_shape=jax.ShapeDtypeStruct((N_PAD, HID), jnp.float32),
    )(parts, xs, dinv_b, bias)


def _tabs_body(h_ref, wd_ref, ws_ref, bd_ref, dt_ref, st_ref):
    h = h_ref[...]
    dt = jnp.dot(h, wd_ref[...], preferred_element_type=jnp.float32) + bd_ref[...]
    st = jnp.dot(h, ws_ref[...], preferred_element_type=jnp.float32)
    dt_ref[...] = dt.astype(jnp.bfloat16)
    st_ref[...] = st.astype(jnp.bfloat16)


def _tc_tabs(h, wdT, wsT, bd):
    grid = (N_PAD // BN,)
    out = jax.ShapeDtypeStruct((N_PAD, 2 * HID), jnp.bfloat16)
    return pl.pallas_call(
        _tabs_body,
        grid=grid,
        in_specs=[
            pl.BlockSpec((BN, HID), lambda i: (i, 0)),
            pl.BlockSpec((HID, 2 * HID), lambda i: (0, 0)),
            pl.BlockSpec((HID, 2 * HID), lambda i: (0, 0)),
            pl.BlockSpec((1, 2 * HID), lambda i: (0, 0)),
        ],
        out_specs=[pl.BlockSpec((BN, 2 * HID), lambda i: (i, 0))] * 2,
        out_shape=[out, out],
    )(h, wdT, wsT, bd)


def _edge_body(gd_ref, gs_ref, ea_ref, we_ref, m_ref):
    ec = jnp.dot(ea_ref[...], we_ref[...], preferred_element_type=jnp.float32)
    p = gd_ref[...].astype(jnp.float32) + gs_ref[...].astype(jnp.float32) + ec
    af = p[:, :HID]
    a2 = p[:, HID:]
    sig = 1.0 / (1.0 + jnp.exp(-af))
    sp = jnp.maximum(a2, 0.0) + jnp.log1p(jnp.exp(-jnp.abs(a2)))
    m_ref[...] = sig * sp


def _tc_edge(gd, gs, ea, weT):
    grid = (N_EDGE // BE,)
    return pl.pallas_call(
        _edge_body,
        grid=grid,
        in_specs=[
            pl.BlockSpec((BE, 2 * HID), lambda i: (i, 0)),
            pl.BlockSpec((BE, 2 * HID), lambda i: (i, 0)),
            pl.BlockSpec((BE, 4), lambda i: (i, 0)),
            pl.BlockSpec((4, 2 * HID), lambda i: (0, 0)),
        ],
        out_specs=pl.BlockSpec((BE, HID), lambda i: (i, 0)),
        out_shape=jax.ShapeDtypeStruct((N_EDGE, HID), jnp.float32),
    )(gd, gs, ea, weT)


def _cg_comb_body(p_ref, h_ref, recip_ref, dinv_ref, o_ref, hs_ref):
    p = p_ref[...]
    mean = (p[0] + p[1]) * recip_ref[...]
    hn = jnp.maximum(mean + h_ref[...], 0.0)
    o_ref[...] = hn
    hs_ref[...] = hn * dinv_ref[...]


def _tc_cg_combine(parts, h, recip_b, dinv_b):
    grid = (N_PAD // BN,)
    out = jax.ShapeDtypeStruct((N_PAD, HID), jnp.float32)
    return pl.pallas_call(
        _cg_comb_body,
        grid=grid,
        in_specs=[
            pl.BlockSpec((NC, BN, HID), lambda i: (0, i, 0)),
            pl.BlockSpec((BN, HID), lambda i: (i, 0)),
            pl.BlockSpec((BN, HID), lambda i: (i, 0)),
            pl.BlockSpec((BN, HID), lambda i: (i, 0)),
        ],
        out_specs=[pl.BlockSpec((BN, HID), lambda i: (i, 0))] * 2,
        out_shape=[out, out],
    )(parts, h, recip_b, dinv_b)


def _final_body(p_ref, hs_ref, dinv_ref, w_ref, b_ref, o_ref):
    p = p_ref[...]
    t = dinv_ref[...] * (p[0] + p[1] + hs_ref[...])
    o_ref[...] = jnp.dot(t, w_ref[...], preferred_element_type=jnp.float32) + b_ref[...]


def _tc_final(parts, hs, dinv_b, w2T, b2p):
    grid = (N_PAD // BN,)
    return pl.pallas_call(
        _final_body,
        grid=grid,
        in_specs=[
            pl.BlockSpec((NC, BN, HID), lambda i: (0, i, 0)),
            pl.BlockSpec((BN, HID), lambda i: (i, 0)),
            pl.BlockSpec((BN, HID), lambda i: (i, 0)),
            pl.BlockSpec((HID, HID), lambda i: (0, 0)),
            pl.BlockSpec((1, HID), lambda i: (0, 0)),
        ],
        out_specs=pl.BlockSpec((BN, HID), lambda i: (i, 0)),
        out_shape=jax.ShapeDtypeStruct((N_PAD, HID), jnp.float32),
    )(parts, hs, dinv_b, w2T, b2p)


def kernel(x, edge_index, edge_attr, W1, b1, Wf1, bf1, Ws1, bs1, Wf2, bf2,
           Ws2, bs2, W2, b2):
    ei = edge_index.astype(jnp.int32)
    src = ei[0]
    dst = ei[1]
    xp = jnp.zeros((N_PAD, x.shape[1]), jnp.float32).at[:N_NODE].set(x)

    cnt_parts = _sc_count(dst)
    dinv_b, recip_b = _tc_deg(cnt_parts)

    xs = _tc_mm_scale(xp, W1.T, dinv_b)
    p1 = _sc_gather_scatter(src, dst, xs)
    h = _tc_gcn_combine(p1, xs, dinv_b, b1.reshape(1, HID))

    def _pack(t):
        return lax.bitcast_convert_type(t.reshape(N_PAD, HID, 2), jnp.int32)

    def _unpack(g):
        return lax.bitcast_convert_type(g, jnp.bfloat16).reshape(N_EDGE, 2 * HID)

    hs = None
    for Wf, bf, Ws, bs in ((Wf1, bf1, Ws1, bs1), (Wf2, bf2, Ws2, bs2)):
        wdT = jnp.concatenate([Wf[:, :HID], Ws[:, :HID]], axis=0).T
        wsT = jnp.concatenate([Wf[:, HID:2 * HID], Ws[:, HID:2 * HID]], axis=0).T
        weT = jnp.concatenate([Wf[:, 2 * HID:], Ws[:, 2 * HID:]], axis=0).T
        bd = jnp.concatenate([bf, bs]).reshape(1, 2 * HID)
        dt, st = _tc_tabs(h, wdT, wsT, bd)
        gd = _sc_gather(dst, _pack(dt))
        gs = _sc_gather(src, _pack(st))
        m = _tc_edge(_unpack(gd), _unpack(gs), edge_attr, weT)
        pm = _sc_scatter(dst, m)
        h, hs = _tc_cg_combine(pm, h, recip_b, dinv_b)

    p2 = _sc_gather_scatter(src, dst, hs)
    w2T = jnp.zeros((HID, HID), jnp.float32).at[:, :2].set(W2.T)
    b2p = jnp.zeros((1, HID), jnp.float32).at[0, :2].set(b2)
    out = _tc_final(p2, hs, dinv_b, w2T, b2p)
    return out[:N_NODE, :2]


# speedup vs baseline: 3.2159x; 3.2159x over previous
import functools

import jax
import jax.numpy as jnp
from jax import lax
from jax.experimental import pallas as pl
from jax.experimental.pallas import tpu as pltpu
from jax.experimental.pallas import tpu_sc as plsc

N_NODE = 10000
N_PAD = 10240
N_EDGE = 320000
HID = 128

NC = 2
NS = 16
NW = NC * NS
EPT = N_EDGE // NW
CH = 128
NFULL = EPT // CH
TAIL = EPT - NFULL * CH
RPT = N_PAD // NS

BN = 1024
BE = 2000


@functools.cache
def _mesh():
    return plsc.VectorSubcoreMesh(
        core_axis_name="c", subcore_axis_name="s", num_cores=NC, num_subcores=NS
    )


def _zero_rows(zb_v, nrows, ncolgrp):
    @pl.loop(0, nrows)
    def _(r):
        @pl.loop(0, ncolgrp)
        def _(j):
            zb_v[r, pl.ds(j * 16, 16)] = jnp.zeros((16,), jnp.float32)


def _zero_acc(zb_v, acc, sid):
    _zero_rows(zb_v, 64, 8)

    @pl.loop(0, RPT // 64)
    def _(k):
        pltpu.sync_copy(zb_v, acc.at[pl.ds(sid * RPT + k * 64, 64)])


def _dump_acc(acc, out_hbm, cid, sid):
    pltpu.sync_copy(acc.at[pl.ds(sid * RPT, RPT)],
                    out_hbm.at[cid, pl.ds(sid * RPT, RPT)])


def _count_body(dst_hbm, ones_hbm, out_hbm, idx_v, idx_t, ones_v, zb_v, acc):
    cid = lax.axis_index("c")
    sid = lax.axis_index("s")
    wid = cid * NS + sid

    pltpu.sync_copy(ones_hbm, ones_v)
    _zero_acc(zb_v, acc, sid)
    plsc.subcore_barrier()
    ebase = wid * EPT

    @pl.loop(0, NFULL)
    def _(c):
        pltpu.sync_copy(dst_hbm.at[pl.ds(ebase + c * CH, CH)], idx_v)
        pltpu.sync_copy(ones_v, acc.at[idx_v], add=True)

    eb = ebase + NFULL * CH
    pltpu.sync_copy(dst_hbm.at[pl.ds(eb, TAIL)], idx_t)
    pltpu.sync_copy(ones_v.at[pl.ds(0, TAIL)], acc.at[idx_t], add=True)

    plsc.subcore_barrier()
    _dump_acc(acc, out_hbm, cid, sid)


def _sc_count(dst):
    f = pl.kernel(
        _count_body,
        out_type=jax.ShapeDtypeStruct((NC, N_PAD, HID), jnp.float32),
        mesh=_mesh(),
        scratch_types=[
            pltpu.VMEM((CH,), jnp.int32),
            pltpu.VMEM((TAIL,), jnp.int32),
            pltpu.VMEM((CH, HID), jnp.float32),
            pltpu.VMEM((64, HID), jnp.float32),
            pltpu.VMEM_SHARED((N_PAD, HID), jnp.float32),
        ],
    )
    return f(dst, jnp.ones((CH, HID), jnp.float32))


def _gather_body(idx_hbm, tab_hbm, out_hbm, idx_v, idx_t, rows_v):
    wid = lax.axis_index("c") * NS + lax.axis_index("s")
    ebase = wid * EPT

    @pl.loop(0, NFULL)
    def _(c):
        eb = ebase + c * CH
        pltpu.sync_copy(idx_hbm.at[pl.ds(eb, CH)], idx_v)
        pltpu.sync_copy(tab_hbm.at[idx_v], rows_v)
        pltpu.sync_copy(rows_v, out_hbm.at[pl.ds(eb, CH)])

    eb = ebase + NFULL * CH
    pltpu.sync_copy(idx_hbm.at[pl.ds(eb, TAIL)], idx_t)
    pltpu.sync_copy(tab_hbm.at[idx_t], rows_v.at[pl.ds(0, TAIL)])
    pltpu.sync_copy(rows_v.at[pl.ds(0, TAIL)], out_hbm.at[pl.ds(eb, TAIL)])


def _sc_gather(idx, tab):
    width = tab.shape[1]
    f = pl.kernel(
        _gather_body,
        out_type=jax.ShapeDtypeStruct((N_EDGE, width), tab.dtype),
        mesh=_mesh(),
        scratch_types=[
            pltpu.VMEM((CH,), jnp.int32),
            pltpu.VMEM((TAIL,), jnp.int32),
            pltpu.VMEM((CH, width), tab.dtype),
        ],
    )
    return f(idx, tab)


def _gather_scatter_body(srcidx_hbm, dstidx_hbm, tab_hbm, out_hbm,
                         isv, idv, ist, idt, rows_v, zb_v, acc):
    cid = lax.axis_index("c")
    sid = lax.axis_index("s")
    wid = cid * NS + sid

    _zero_acc(zb_v, acc, sid)
    plsc.subcore_barrier()
    ebase = wid * EPT

    @pl.loop(0, NFULL)
    def _(c):
        eb = ebase + c * CH
        pltpu.sync_copy(srcidx_hbm.at[pl.ds(eb, CH)], isv)
        pltpu.sync_copy(dstidx_hbm.at[pl.ds(eb, CH)], idv)
        pltpu.sync_copy(tab_hbm.at[isv], rows_v)
        pltpu.sync_copy(rows_v, acc.at[idv], add=True)

    eb = ebase + NFULL * CH
    pltpu.sync_copy(srcidx_hbm.at[pl.ds(eb, TAIL)], ist)
    pltpu.sync_copy(dstidx_hbm.at[pl.ds(eb, TAIL)], idt)
    pltpu.sync_copy(tab_hbm.at[ist], rows_v.at[pl.ds(0, TAIL)])
    pltpu.sync_copy(rows_v.at[pl.ds(0, TAIL)], acc.at[idt], add=True)

    plsc.subcore_barrier()
    _dump_acc(acc, out_hbm, cid, sid)


def _sc_gather_scatter(src, dst, tab):
    f = pl.kernel(
        _gather_scatter_body,
        out_type=jax.ShapeDtypeStruct((NC, N_PAD, HID), jnp.float32),
        mesh=_mesh(),
        scratch_types=[
            pltpu.VMEM((CH,), jnp.int32),
            pltpu.VMEM((CH,), jnp.int32),
            pltpu.VMEM((TAIL,), jnp.int32),
            pltpu.VMEM((TAIL,), jnp.int32),
            pltpu.VMEM((CH, HID), jnp.float32),
            pltpu.VMEM((64, HID), jnp.float32),
            pltpu.VMEM_SHARED((N_PAD, HID), jnp.float32),
        ],
    )
    return f(src, dst, tab)


def _scatter_body(idx_hbm, val_hbm, out_hbm, idx_v, idx_t, val_v, zb_v, acc):
    cid = lax.axis_index("c")
    sid = lax.axis_index("s")
    wid = cid * NS + sid

    _zero_acc(zb_v, acc, sid)
    plsc.subcore_barrier()
    ebase = wid * EPT

    @pl.loop(0, NFULL)
    def _(c):
        eb = ebase + c * CH
        pltpu.sync_copy(idx_hbm.at[pl.ds(eb, CH)], idx_v)
        pltpu.sync_copy(val_hbm.at[pl.ds(eb, CH)], val_v)
        pltpu.sync_copy(val_v, acc.at[idx_v], add=True)

    eb = ebase + NFULL * CH
    pltpu.sync_copy(idx_hbm.at[pl.ds(eb, TAIL)], idx_t)
    pltpu.sync_copy(val_hbm.at[pl.ds(eb, TAIL)], val_v.at[pl.ds(0, TAIL)])
    pltpu.sync_copy(val_v.at[pl.ds(0, TAIL)], acc.at[idx_t], add=True)

    plsc.subcore_barrier()
    _dump_acc(acc, out_hbm, cid, sid)


def _sc_scatter(idx, vals):
    f = pl.kernel(
        _scatter_body,
        out_type=jax.ShapeDtypeStruct((NC, N_PAD, HID), jnp.float32),
        mesh=_mesh(),
        scratch_types=[
            pltpu.VMEM((CH,), jnp.int32),
            pltpu.VMEM((TAIL,), jnp.int32),
            pltpu.VMEM((CH, HID), jnp.float32),
            pltpu.VMEM((64, HID), jnp.float32),
            pltpu.VMEM_SHARED((N_PAD, HID), jnp.float32),
        ],
    )
    return f(idx, vals)


def _pack_pair(f, s):
    himask = jnp.uint32(0xFFFF0000)
    uf = lax.bitcast_convert_type(f, jnp.uint32)
    uf = uf + 0x7FFF + ((uf >> 16) & 1)
    us = lax.bitcast_convert_type(s, jnp.uint32)
    us = us + 0x7FFF + ((us >> 16) & 1)
    packed = (uf & himask) | (us >> 16)
    return lax.bitcast_convert_type(packed, jnp.int32)


def _unpack_pair(g):
    u = lax.bitcast_convert_type(g, jnp.uint32)
    hi = lax.bitcast_convert_type(u & jnp.uint32(0xFFFF0000), jnp.float32)
    lo = lax.bitcast_convert_type(u << 16, jnp.float32)
    return hi, lo


def _deg_body(p_ref, dinv_ref, recip_ref):
    p = p_ref[...]
    cnt = (p[0] + p[1])[:, 0:1]
    dinv = lax.rsqrt(cnt + 1.0)
    recip = 1.0 / jnp.maximum(cnt, 1.0)
    dinv_ref[...] = jnp.broadcast_to(dinv, dinv_ref.shape)
    recip_ref[...] = jnp.broadcast_to(recip, recip_ref.shape)


def _tc_deg(parts):
    grid = (N_PAD // BN,)
    out = jax.ShapeDtypeStruct((N_PAD, HID), jnp.float32)
    return pl.pallas_call(
        _deg_body,
        grid=grid,
        in_specs=[pl.BlockSpec((NC, BN, HID), lambda i: (0, i, 0))],
        out_specs=[pl.BlockSpec((BN, HID), lambda i: (i, 0))] * 2,
        out_shape=[out, out],
    )(parts)


def _mm_scale_body(x_ref, w_ref, s_ref, o_ref):
    xw = jnp.dot(x_ref[...], w_ref[...], preferred_element_type=jnp.float32)
    o_ref[...] = xw * s_ref[...]


def _tc_mm_scale(x, w, s):
    grid = (N_PAD // BN,)
    return pl.pallas_call(
        _mm_scale_body,
        grid=grid,
        in_specs=[
            pl.BlockSpec((BN, x.shape[1]), lambda i: (i, 0)),
            pl.BlockSpec(w.shape, lambda i: (0, 0)),
            pl.BlockSpec((BN, HID), lambda i: (i, 0)),
        ],
        out_specs=pl.BlockSpec((BN, w.shape[1]), lambda i: (i, 0)),
        out_shape=jax.ShapeDtypeStruct((N_PAD, w.shape[1]), jnp.float32),
    )(x, w, s)


def _gcn_comb_body(p_ref, xs_ref, dinv_ref, b_ref, o_ref):
    p = p_ref[...]
    s = p[0] + p[1] + xs_ref[...]
    o_ref[...] = jnp.maximum(dinv_ref[...] * s + b_ref[...], 0.0)


def _tc_gcn_combine(parts, xs, dinv_b, bias):
    grid = (N_PAD // BN,)
    return pl.pallas_call(
        _gcn_comb_body,
        grid=grid,
        in_specs=[
            pl.BlockSpec((NC, BN, HID), lambda i: (0, i, 0)),
            pl.BlockSpec((BN, HID), lambda i: (i, 0)),
            pl.BlockSpec((BN, HID), lambda i: (i, 0)),
            pl.BlockSpec((1, HID), lambda i: (0, 0)),
        ],
        out_specs=pl.BlockSpec((BN, HID), lambda i: (i, 0)),
        out_shape=jax.ShapeDtypeStruct((N_PAD, HID), jnp.float32),
    )(parts, xs, dinv_b, bias)


def _tabs_body(h_ref, wfd_ref, wsd_ref, wfs_ref, wss_ref, bf_ref, bs_ref,
               dt_ref, st_ref):
    h = h_ref[...]
    fd = jnp.dot(h, wfd_ref[...], preferred_element_type=jnp.float32) + bf_ref[...]
    sd = jnp.dot(h, wsd_ref[...], preferred_element_type=jnp.float32) + bs_ref[...]
    gs = jnp.dot(h, wfs_ref[...], preferred_element_type=jnp.float32)
    ts = jnp.dot(h, wss_ref[...], preferred_element_type=jnp.float32)
    dt_ref[...] = _pack_pair(fd, sd)
    st_ref[...] = _pack_pair(gs, ts)


def _tc_tabs(h, wfdT, wsdT, wfsT, wssT, bf, bs):
    grid = (N_PAD // BN,)
    out = jax.ShapeDtypeStruct((N_PAD, HID), jnp.int32)
    wspec = pl.BlockSpec((HID, HID), lambda i: (0, 0))
    bspec = pl.BlockSpec((1, HID), lambda i: (0, 0))
    return pl.pallas_call(
        _tabs_body,
        grid=grid,
        in_specs=[pl.BlockSpec((BN, HID), lambda i: (i, 0)),
                  wspec, wspec, wspec, wspec, bspec, bspec],
        out_specs=[pl.BlockSpec((BN, HID), lambda i: (i, 0))] * 2,
        out_shape=[out, out],
    )(h, wfdT, wsdT, wfsT, wssT, bf, bs)


def _edge_body(gd_ref, gs_ref, ea_ref, we_ref, m_ref):
    ec = jnp.dot(ea_ref[...], we_ref[...], preferred_element_type=jnp.float32)
    fd, sd = _unpack_pair(gd_ref[...])
    fs, ss = _unpack_pair(gs_ref[...])
    af = fd + fs + ec[:, :HID]
    a2 = sd + ss + ec[:, HID:]
    sig = 1.0 / (1.0 + jnp.exp(-af))
    sp = jnp.maximum(a2, 0.0) + jnp.log1p(jnp.exp(-jnp.abs(a2)))
    m_ref[...] = sig * sp


def _tc_edge(gd, gs, ea, weT):
    grid = (N_EDGE // BE,)
    return pl.pallas_call(
        _edge_body,
        grid=grid,
        in_specs=[
            pl.BlockSpec((BE, HID), lambda i: (i, 0)),
            pl.BlockSpec((BE, HID), lambda i: (i, 0)),
            pl.BlockSpec((BE, 4), lambda i: (i, 0)),
            pl.BlockSpec((4, 2 * HID), lambda i: (0, 0)),
        ],
        out_specs=pl.BlockSpec((BE, HID), lambda i: (i, 0)),
        out_shape=jax.ShapeDtypeStruct((N_EDGE, HID), jnp.float32),
    )(gd, gs, ea, weT)


def _cg_comb_body(p_ref, h_ref, recip_ref, dinv_ref, o_ref, hs_ref):
    p = p_ref[...]
    mean = (p[0] + p[1]) * recip_ref[...]
    hn = jnp.maximum(mean + h_ref[...], 0.0)
    o_ref[...] = hn
    hs_ref[...] = hn * dinv_ref[...]


def _tc_cg_combine(parts, h, recip_b, dinv_b):
    grid = (N_PAD // BN,)
    out = jax.ShapeDtypeStruct((N_PAD, HID), jnp.float32)
    return pl.pallas_call(
        _cg_comb_body,
        grid=grid,
        in_specs=[
            pl.BlockSpec((NC, BN, HID), lambda i: (0, i, 0)),
            pl.BlockSpec((BN, HID), lambda i: (i, 0)),
            pl.BlockSpec((BN, HID), lambda i: (i, 0)),
            pl.BlockSpec((BN, HID), lambda i: (i, 0)),
        ],
        out_specs=[pl.BlockSpec((BN, HID), lambda i: (i, 0))] * 2,
        out_shape=[out, out],
    )(parts, h, recip_b, dinv_b)


def _final_body(p_ref, hs_ref, dinv_ref, w_ref, b_ref, o_ref):
    p = p_ref[...]
    t = dinv_ref[...] * (p[0] + p[1] + hs_ref[...])
    o_ref[...] = jnp.dot(t, w_ref[...], preferred_element_type=jnp.float32) + b_ref[...]


def _tc_final(parts, hs, dinv_b, w2T, b2p):
    grid = (N_PAD // BN,)
    return pl.pallas_call(
        _final_body,
        grid=grid,
        in_specs=[
            pl.BlockSpec((NC, BN, HID), lambda i: (0, i, 0)),
            pl.BlockSpec((BN, HID), lambda i: (i, 0)),
            pl.BlockSpec((BN, HID), lambda i: (i, 0)),
            pl.BlockSpec((HID, HID), lambda i: (0, 0)),
            pl.BlockSpec((1, HID), lambda i: (0, 0)),
        ],
        out_specs=pl.BlockSpec((BN, HID), lambda i: (i, 0)),
        out_shape=jax.ShapeDtypeStruct((N_PAD, HID), jnp.float32),
    )(parts, hs, dinv_b, w2T, b2p)


def kernel(x, edge_index, edge_attr, W1, b1, Wf1, bf1, Ws1, bs1, Wf2, bf2,
           Ws2, bs2, W2, b2):
    ei = edge_index.astype(jnp.int32)
    src = ei[0]
    dst = ei[1]
    xp = jnp.zeros((N_PAD, x.shape[1]), jnp.float32).at[:N_NODE].set(x)

    cnt_parts = _sc_count(dst)
    dinv_b, recip_b = _tc_deg(cnt_parts)

    xs = _tc_mm_scale(xp, W1.T, dinv_b)
    p1 = _sc_gather_scatter(src, dst, xs)
    h = _tc_gcn_combine(p1, xs, dinv_b, b1.reshape(1, HID))

    hs = None
    for Wf, bf, Ws, bs in ((Wf1, bf1, Ws1, bs1), (Wf2, bf2, Ws2, bs2)):
        wfdT = Wf[:, :HID].T
        wsdT = Ws[:, :HID].T
        wfsT = Wf[:, HID:2 * HID].T
        wssT = Ws[:, HID:2 * HID].T
        weT = jnp.concatenate([Wf[:, 2 * HID:], Ws[:, 2 * HID:]], axis=0).T
        dt, st = _tc_tabs(h, wfdT, wsdT, wfsT, wssT,
                          bf.reshape(1, HID), bs.reshape(1, HID))
        gd = _sc_gather(dst, dt)
        gs = _sc_gather(src, st)
        m = _tc_edge(gd, gs, edge_attr, weT)
        pm = _sc_scatter(dst, m)
        h, hs = _tc_cg_combine(pm, h, recip_b, dinv_b)

    p2 = _sc_gather_scatter(src, dst, hs)
    w2T = jnp.zeros((HID, HID), jnp.float32).at[:, :2].set(W2.T)
    b2p = jnp.zeros((1, HID), jnp.float32).at[0, :2].set(b2)
    out = _tc_final(p2, hs, dinv_b, w2T, b2p)
    return out[:N_NODE, :2]


# speedup vs baseline: 4.3664x; 1.3577x over previous
import functools

import jax
import jax.numpy as jnp
from jax import lax
from jax.experimental import pallas as pl
from jax.experimental.pallas import tpu as pltpu
from jax.experimental.pallas import tpu_sc as plsc

N_NODE = 10000
N_PAD = 10240
N_EDGE = 320000
HID = 128

NC = 2
NS = 16
NW = NC * NS
EPT = N_EDGE // NW
CH = 128
NFULL = EPT // CH
TAIL = EPT - NFULL * CH
RPT = N_PAD // NS

BN = 1024
BE = 2000


@functools.cache
def _mesh():
    return plsc.VectorSubcoreMesh(
        core_axis_name="c", subcore_axis_name="s", num_cores=NC, num_subcores=NS
    )


K = 3
NGG = NFULL // K
KS = 1
NGS = NFULL // KS


def _zero_rows(zb_v, nrows, ncolgrp):
    @pl.loop(0, nrows)
    def _(r):
        @pl.loop(0, ncolgrp)
        def _(j):
            zb_v[r, pl.ds(j * 16, 16)] = jnp.zeros((16,), jnp.float32)


def _zero_acc(zb_v, acc, sid):
    _zero_rows(zb_v, 32, 8)

    @pl.loop(0, RPT // 32)
    def _(k):
        pltpu.sync_copy(zb_v, acc.at[pl.ds(sid * RPT + k * 32, 32)])


def _dump_acc(acc, out_hbm, cid, sid):
    pltpu.sync_copy(acc.at[pl.ds(sid * RPT, RPT)],
                    out_hbm.at[cid, pl.ds(sid * RPT, RPT)])


def _gather_body(idx_hbm, tab_hbm, out_hbm,
                 i00, i01, i02, i10, i11, i12, rows0, rows1, idxt,
                 si0, si1, sg0, sg1, sw0, sw1):
    wid = lax.axis_index("c") * NS + lax.axis_index("s")
    ebase = wid * EPT
    isv = ((i00, i01, i02), (i10, i11, i12))
    rows = (rows0, rows1)
    si = (si0, si1)
    sg = (sg0, sg1)
    sw = (sw0, sw1)

    def load_idx(g, b):
        for j in range(K):
            pltpu.async_copy(idx_hbm.at[pl.ds(ebase + (g * K + j) * CH, CH)],
                             isv[b][j], si[b])

    def wait_idx(g, b):
        for j in range(K):
            pltpu.make_async_copy(
                idx_hbm.at[pl.ds(ebase + (g * K + j) * CH, CH)],
                isv[b][j], si[b]).wait()

    load_idx(0, 0)

    def sub(g, b):
        ebg = ebase + g * K * CH
        wait_idx(g, b)

        @pl.when(g >= 2)
        def _():
            ebw = ebase + (g - 2) * K * CH
            pltpu.make_async_copy(rows[b], out_hbm.at[pl.ds(ebw, K * CH)],
                                  sw[b]).wait()

        for j in range(K):
            pltpu.async_copy(tab_hbm.at[isv[b][j]],
                             rows[b].at[pl.ds(j * CH, CH)], sg[b])
        for j in range(K):
            pltpu.make_async_copy(tab_hbm.at[isv[b][j]],
                                  rows[b].at[pl.ds(j * CH, CH)], sg[b]).wait()

        pltpu.async_copy(rows[b], out_hbm.at[pl.ds(ebg, K * CH)], sw[b])

        @pl.when(g + 1 < NGG)
        def _():
            load_idx(g + 1, 1 - b)

    @pl.loop(0, NGG // 2)
    def _(k2):
        sub(2 * k2, 0)
        sub(2 * k2 + 1, 1)

    pltpu.make_async_copy(
        rows[0], out_hbm.at[pl.ds(ebase + (NGG - 2) * K * CH, K * CH)],
        sw[0]).wait()
    pltpu.make_async_copy(
        rows[1], out_hbm.at[pl.ds(ebase + (NGG - 1) * K * CH, K * CH)],
        sw[1]).wait()

    ebt = ebase + NFULL * CH
    pltpu.sync_copy(idx_hbm.at[pl.ds(ebt, TAIL)], idxt)
    pltpu.sync_copy(tab_hbm.at[idxt], rows0.at[pl.ds(0, TAIL)])
    pltpu.sync_copy(rows0.at[pl.ds(0, TAIL)], out_hbm.at[pl.ds(ebt, TAIL)])


def _sc_gather(idx, tab):
    width = tab.shape[1]
    f = pl.kernel(
        _gather_body,
        out_type=jax.ShapeDtypeStruct((N_EDGE, width), tab.dtype),
        mesh=_mesh(),
        scratch_types=[
            pltpu.VMEM((CH,), jnp.int32),
            pltpu.VMEM((CH,), jnp.int32),
            pltpu.VMEM((CH,), jnp.int32),
            pltpu.VMEM((CH,), jnp.int32),
            pltpu.VMEM((CH,), jnp.int32),
            pltpu.VMEM((CH,), jnp.int32),
            pltpu.VMEM((K * CH, width), tab.dtype),
            pltpu.VMEM((K * CH, width), tab.dtype),
            pltpu.VMEM((TAIL,), jnp.int32),
        ] + [pltpu.SemaphoreType.DMA] * 6,
    )
    return f(idx, tab)


def _scatter_body(idx_hbm, val_hbm, out_hbm, isv0, isv1, vals0, vals1, idxt,
                  zb_v, acc, sl0, sl1, ss0, ss1):
    cid = lax.axis_index("c")
    sid = lax.axis_index("s")
    wid = cid * NS + sid
    ebase = wid * EPT
    isv = (isv0, isv1)
    vals = (vals0, vals1)
    sl = (sl0, sl1)
    ss = (ss0, ss1)

    _zero_acc(zb_v, acc, sid)
    plsc.subcore_barrier()

    def load(g, b):
        ebg = ebase + g * CH
        pltpu.async_copy(idx_hbm.at[pl.ds(ebg, CH)], isv[b], sl[b])
        pltpu.async_copy(val_hbm.at[pl.ds(ebg, CH)], vals[b], sl[b])

    def wait_load(g, b):
        ebg = ebase + g * CH
        pltpu.make_async_copy(idx_hbm.at[pl.ds(ebg, CH)], isv[b], sl[b]).wait()
        pltpu.make_async_copy(val_hbm.at[pl.ds(ebg, CH)], vals[b], sl[b]).wait()

    load(0, 0)

    def sub(g, b):
        wait_load(g, b)

        @pl.when(g >= 1)
        def _():
            pltpu.make_async_copy(vals[1 - b], acc.at[isv[1 - b]],
                                  ss[1 - b]).wait()

        pltpu.async_copy(vals[b], acc.at[isv[b]], ss[b], add=True)

        @pl.when(g + 1 < NGS)
        def _():
            load(g + 1, 1 - b)

    @pl.loop(0, NGS // 2)
    def _(k2):
        sub(2 * k2, 0)
        sub(2 * k2 + 1, 1)

    pltpu.make_async_copy(vals[1], acc.at[isv[1]], ss[1]).wait()

    ebt = ebase + NFULL * CH
    pltpu.sync_copy(idx_hbm.at[pl.ds(ebt, TAIL)], idxt)
    pltpu.sync_copy(val_hbm.at[pl.ds(ebt, TAIL)], vals0.at[pl.ds(0, TAIL)])
    pltpu.sync_copy(vals0.at[pl.ds(0, TAIL)], acc.at[idxt], add=True)

    plsc.subcore_barrier()
    _dump_acc(acc, out_hbm, cid, sid)


def _sc_scatter(idx, vals):
    f = pl.kernel(
        _scatter_body,
        out_type=jax.ShapeDtypeStruct((NC, N_PAD, HID), jnp.float32),
        mesh=_mesh(),
        scratch_types=[
            pltpu.VMEM((CH,), jnp.int32),
            pltpu.VMEM((CH,), jnp.int32),
            pltpu.VMEM((CH, HID), jnp.float32),
            pltpu.VMEM((CH, HID), jnp.float32),
            pltpu.VMEM((TAIL,), jnp.int32),
            pltpu.VMEM((32, HID), jnp.float32),
            pltpu.VMEM_SHARED((N_PAD, HID), jnp.float32),
        ] + [pltpu.SemaphoreType.DMA] * 4,
    )
    return f(idx, vals)


def _count_body(dst_hbm, ones_hbm, out_hbm, isv0, isv1, idxt, ones_v, zb_v,
                acc, sl0, sl1, ss0, ss1):
    cid = lax.axis_index("c")
    sid = lax.axis_index("s")
    wid = cid * NS + sid
    ebase = wid * EPT
    isv = (isv0, isv1)
    sl = (sl0, sl1)
    ss = (ss0, ss1)

    pltpu.sync_copy(ones_hbm, ones_v)
    _zero_acc(zb_v, acc, sid)
    plsc.subcore_barrier()

    pltpu.async_copy(dst_hbm.at[pl.ds(ebase, CH)], isv[0], sl[0])

    def sub(g, b):
        ebg = ebase + g * CH
        pltpu.make_async_copy(dst_hbm.at[pl.ds(ebg, CH)], isv[b], sl[b]).wait()

        @pl.when(g >= 1)
        def _():
            pltpu.make_async_copy(ones_v, acc.at[isv[1 - b]], ss[1 - b]).wait()

        pltpu.async_copy(ones_v, acc.at[isv[b]], ss[b], add=True)

        @pl.when(g + 1 < NGS)
        def _():
            pltpu.async_copy(dst_hbm.at[pl.ds(ebg + CH, CH)], isv[1 - b],
                             sl[1 - b])

    @pl.loop(0, NGS // 2)
    def _(k2):
        sub(2 * k2, 0)
        sub(2 * k2 + 1, 1)

    pltpu.make_async_copy(ones_v, acc.at[isv[1]], ss[1]).wait()

    ebt = ebase + NFULL * CH
    pltpu.sync_copy(dst_hbm.at[pl.ds(ebt, TAIL)], idxt)
    pltpu.sync_copy(ones_v.at[pl.ds(0, TAIL)], acc.at[idxt], add=True)

    plsc.subcore_barrier()
    _dump_acc(acc, out_hbm, cid, sid)


def _sc_count(dst):
    f = pl.kernel(
        _count_body,
        out_type=jax.ShapeDtypeStruct((NC, N_PAD, HID), jnp.float32),
        mesh=_mesh(),
        scratch_types=[
            pltpu.VMEM((CH,), jnp.int32),
            pltpu.VMEM((CH,), jnp.int32),
            pltpu.VMEM((TAIL,), jnp.int32),
            pltpu.VMEM((CH, HID), jnp.float32),
            pltpu.VMEM((32, HID), jnp.float32),
            pltpu.VMEM_SHARED((N_PAD, HID), jnp.float32),
        ] + [pltpu.SemaphoreType.DMA] * 4,
    )
    return f(dst, jnp.ones((CH, HID), jnp.float32))


def _gather_scatter_body(src_hbm, dst_hbm, tab_hbm, out_hbm, iss0, iss1,
                         idd0, idd1, rows0, rows1, ist, idt, zb_v, acc,
                         sis0, sis1, sd0, sd1, sg0, sg1, ssc0, ssc1):
    cid = lax.axis_index("c")
    sid = lax.axis_index("s")
    wid = cid * NS + sid
    ebase = wid * EPT
    iss = (iss0, iss1)
    idd = (idd0, idd1)
    rows = (rows0, rows1)
    sis = (sis0, sis1)
    sdm = (sd0, sd1)
    sg = (sg0, sg1)
    ssc = (ssc0, ssc1)

    _zero_acc(zb_v, acc, sid)
    plsc.subcore_barrier()

    pltpu.async_copy(src_hbm.at[pl.ds(ebase, CH)], iss[0], sis[0])

    def sub(g, b):
        ebg = ebase + g * CH
        pltpu.make_async_copy(src_hbm.at[pl.ds(ebg, CH)], iss[b],
                              sis[b]).wait()

        @pl.when(g >= 2)
        def _():
            pltpu.make_async_copy(rows[b], acc.at[idd[b]], ssc[b]).wait()

        pltpu.async_copy(dst_hbm.at[pl.ds(ebg, CH)], idd[b], sdm[b])

        pltpu.async_copy(tab_hbm.at[iss[b]], rows[b], sg[b])
        pltpu.make_async_copy(tab_hbm.at[iss[b]], rows[b], sg[b]).wait()

        @pl.when(g + 1 < NGS)
        def _():
            pltpu.async_copy(src_hbm.at[pl.ds(ebg + CH, CH)], iss[1 - b],
                             sis[1 - b])

        pltpu.make_async_copy(dst_hbm.at[pl.ds(ebg, CH)], idd[b],
                              sdm[b]).wait()

        pltpu.async_copy(rows[b], acc.at[idd[b]], ssc[b], add=True)

    @pl.loop(0, NGS // 2)
    def _(k2):
        sub(2 * k2, 0)
        sub(2 * k2 + 1, 1)

    pltpu.make_async_copy(rows[0], acc.at[idd[0]], ssc[0]).wait()
    pltpu.make_async_copy(rows[1], acc.at[idd[1]], ssc[1]).wait()

    ebt = ebase + NFULL * CH
    pltpu.sync_copy(src_hbm.at[pl.ds(ebt, TAIL)], ist)
    pltpu.sync_copy(dst_hbm.at[pl.ds(ebt, TAIL)], idt)
    pltpu.sync_copy(tab_hbm.at[ist], rows0.at[pl.ds(0, TAIL)])
    pltpu.sync_copy(rows0.at[pl.ds(0, TAIL)], acc.at[idt], add=True)

    plsc.subcore_barrier()
    _dump_acc(acc, out_hbm, cid, sid)


def _sc_gather_scatter(src, dst, tab):
    f = pl.kernel(
        _gather_scatter_body,
        out_type=jax.ShapeDtypeStruct((NC, N_PAD, HID), jnp.float32),
        mesh=_mesh(),
        scratch_types=[
            pltpu.VMEM((CH,), jnp.int32),
            pltpu.VMEM((CH,), jnp.int32),
            pltpu.VMEM((CH,), jnp.int32),
            pltpu.VMEM((CH,), jnp.int32),
            pltpu.VMEM((CH, HID), jnp.float32),
            pltpu.VMEM((CH, HID), jnp.float32),
            pltpu.VMEM((TAIL,), jnp.int32),
            pltpu.VMEM((TAIL,), jnp.int32),
            pltpu.VMEM((32, HID), jnp.float32),
            pltpu.VMEM_SHARED((N_PAD, HID), jnp.float32),
        ] + [pltpu.SemaphoreType.DMA] * 8,
    )
    return f(src, dst, tab)


def _pack_pair(f, s):
    himask = jnp.uint32(0xFFFF0000)
    uf = lax.bitcast_convert_type(f, jnp.uint32)
    uf = uf + 0x7FFF + ((uf >> 16) & 1)
    us = lax.bitcast_convert_type(s, jnp.uint32)
    us = us + 0x7FFF + ((us >> 16) & 1)
    packed = (uf & himask) | (us >> 16)
    return lax.bitcast_convert_type(packed, jnp.int32)


def _unpack_pair(g):
    u = lax.bitcast_convert_type(g, jnp.uint32)
    hi = lax.bitcast_convert_type(u & jnp.uint32(0xFFFF0000), jnp.float32)
    lo = lax.bitcast_convert_type(u << 16, jnp.float32)
    return hi, lo


def _deg_body(p_ref, dinv_ref, recip_ref):
    p = p_ref[...]
    cnt = (p[0] + p[1])[:, 0:1]
    dinv = lax.rsqrt(cnt + 1.0)
    recip = 1.0 / jnp.maximum(cnt, 1.0)
    dinv_ref[...] = jnp.broadcast_to(dinv, dinv_ref.shape)
    recip_ref[...] = jnp.broadcast_to(recip, recip_ref.shape)


def _tc_deg(parts):
    grid = (N_PAD // BN,)
    out = jax.ShapeDtypeStruct((N_PAD, HID), jnp.float32)
    return pl.pallas_call(
        _deg_body,
        grid=grid,
        in_specs=[pl.BlockSpec((NC, BN, HID), lambda i: (0, i, 0))],
        out_specs=[pl.BlockSpec((BN, HID), lambda i: (i, 0))] * 2,
        out_shape=[out, out],
    )(parts)


def _mm_scale_body(x_ref, w_ref, s_ref, o_ref):
    xw = jnp.dot(x_ref[...], w_ref[...], preferred_element_type=jnp.float32)
    o_ref[...] = xw * s_ref[...]


def _tc_mm_scale(x, w, s):
    grid = (N_PAD // BN,)
    return pl.pallas_call(
        _mm_scale_body,
        grid=grid,
        in_specs=[
            pl.BlockSpec((BN, x.shape[1]), lambda i: (i, 0)),
            pl.BlockSpec(w.shape, lambda i: (0, 0)),
            pl.BlockSpec((BN, HID), lambda i: (i, 0)),
        ],
        out_specs=pl.BlockSpec((BN, w.shape[1]), lambda i: (i, 0)),
        out_shape=jax.ShapeDtypeStruct((N_PAD, w.shape[1]), jnp.float32),
    )(x, w, s)


def _gcn_comb_body(p_ref, xs_ref, dinv_ref, b_ref, o_ref):
    p = p_ref[...]
    s = p[0] + p[1] + xs_ref[...]
    o_ref[...] = jnp.maximum(dinv_ref[...] * s + b_ref[...], 0.0)


def _tc_gcn_combine(parts, xs, dinv_b, bias):
    grid = (N_PAD // BN,)
    return pl.pallas_call(
        _gcn_comb_body,
        grid=grid,
        in_specs=[
            pl.BlockSpec((NC, BN, HID), lambda i: (0, i, 0)),
            pl.BlockSpec((BN, HID), lambda i: (i, 0)),
            pl.BlockSpec((BN, HID), lambda i: (i, 0)),
            pl.BlockSpec((1, HID), lambda i: (0, 0)),
        ],
        out_specs=pl.BlockSpec((BN, HID), lambda i: (i, 0)),
        out_shape=jax.ShapeDtypeStruct((N_PAD, HID), jnp.float32),
    )(parts, xs, dinv_b, bias)


def _tabs_body(h_ref, wfd_ref, wsd_ref, wfs_ref, wss_ref, bf_ref, bs_ref,
               dt_ref, st_ref):
    h = h_ref[...]
    fd = jnp.dot(h, wfd_ref[...], preferred_element_type=jnp.float32) + bf_ref[...]
    sd = jnp.dot(h, wsd_ref[...], preferred_element_type=jnp.float32) + bs_ref[...]
    gs = jnp.dot(h, wfs_ref[...], preferred_element_type=jnp.float32)
    ts = jnp.dot(h, wss_ref[...], preferred_element_type=jnp.float32)
    dt_ref[...] = _pack_pair(fd, sd)
    st_ref[...] = _pack_pair(gs, ts)


def _tc_tabs(h, wfdT, wsdT, wfsT, wssT, bf, bs):
    grid = (N_PAD // BN,)
    out = jax.ShapeDtypeStruct((N_PAD, HID), jnp.int32)
    wspec = pl.BlockSpec((HID, HID), lambda i: (0, 0))
    bspec = pl.BlockSpec((1, HID), lambda i: (0, 0))
    return pl.pallas_call(
        _tabs_body,
        grid=grid,
        in_specs=[pl.BlockSpec((BN, HID), lambda i: (i, 0)),
                  wspec, wspec, wspec, wspec, bspec, bspec],
        out_specs=[pl.BlockSpec((BN, HID), lambda i: (i, 0))] * 2,
        out_shape=[out, out],
    )(h, wfdT, wsdT, wfsT, wssT, bf, bs)


def _edge_body(gd_ref, gs_ref, ea_ref, we_ref, m_ref):
    ec = jnp.dot(ea_ref[...], we_ref[...], preferred_element_type=jnp.float32)
    fd, sd = _unpack_pair(gd_ref[...])
    fs, ss = _unpack_pair(gs_ref[...])
    af = fd + fs + ec[:, :HID]
    a2 = sd + ss + ec[:, HID:]
    sig = 1.0 / (1.0 + jnp.exp(-af))
    sp = jnp.maximum(a2, 0.0) + jnp.log1p(jnp.exp(-jnp.abs(a2)))
    m_ref[...] = sig * sp


def _tc_edge(gd, gs, ea, weT):
    grid = (N_EDGE // BE,)
    return pl.pallas_call(
        _edge_body,
        grid=grid,
        in_specs=[
            pl.BlockSpec((BE, HID), lambda i: (i, 0)),
            pl.BlockSpec((BE, HID), lambda i: (i, 0)),
            pl.BlockSpec((BE, 4), lambda i: (i, 0)),
            pl.BlockSpec((4, 2 * HID), lambda i: (0, 0)),
        ],
        out_specs=pl.BlockSpec((BE, HID), lambda i: (i, 0)),
        out_shape=jax.ShapeDtypeStruct((N_EDGE, HID), jnp.float32),
    )(gd, gs, ea, weT)


def _cg_comb_body(p_ref, h_ref, recip_ref, dinv_ref, o_ref, hs_ref):
    p = p_ref[...]
    mean = (p[0] + p[1]) * recip_ref[...]
    hn = jnp.maximum(mean + h_ref[...], 0.0)
    o_ref[...] = hn
    hs_ref[...] = hn * dinv_ref[...]


def _tc_cg_combine(parts, h, recip_b, dinv_b):
    grid = (N_PAD // BN,)
    out = jax.ShapeDtypeStruct((N_PAD, HID), jnp.float32)
    return pl.pallas_call(
        _cg_comb_body,
        grid=grid,
        in_specs=[
            pl.BlockSpec((NC, BN, HID), lambda i: (0, i, 0)),
            pl.BlockSpec((BN, HID), lambda i: (i, 0)),
            pl.BlockSpec((BN, HID), lambda i: (i, 0)),
            pl.BlockSpec((BN, HID), lambda i: (i, 0)),
        ],
        out_specs=[pl.BlockSpec((BN, HID), lambda i: (i, 0))] * 2,
        out_shape=[out, out],
    )(parts, h, recip_b, dinv_b)


def _final_body(p_ref, hs_ref, dinv_ref, w_ref, b_ref, o_ref):
    p = p_ref[...]
    t = dinv_ref[...] * (p[0] + p[1] + hs_ref[...])
    o_ref[...] = jnp.dot(t, w_ref[...], preferred_element_type=jnp.float32) + b_ref[...]


def _tc_final(parts, hs, dinv_b, w2T, b2p):
    grid = (N_PAD // BN,)
    return pl.pallas_call(
        _final_body,
        grid=grid,
        in_specs=[
            pl.BlockSpec((NC, BN, HID), lambda i: (0, i, 0)),
            pl.BlockSpec((BN, HID), lambda i: (i, 0)),
            pl.BlockSpec((BN, HID), lambda i: (i, 0)),
            pl.BlockSpec((HID, HID), lambda i: (0, 0)),
            pl.BlockSpec((1, HID), lambda i: (0, 0)),
        ],
        out_specs=pl.BlockSpec((BN, HID), lambda i: (i, 0)),
        out_shape=jax.ShapeDtypeStruct((N_PAD, HID), jnp.float32),
    )(parts, hs, dinv_b, w2T, b2p)


def kernel(x, edge_index, edge_attr, W1, b1, Wf1, bf1, Ws1, bs1, Wf2, bf2,
           Ws2, bs2, W2, b2):
    ei = edge_index.astype(jnp.int32)
    src = ei[0]
    dst = ei[1]
    xp = jnp.zeros((N_PAD, x.shape[1]), jnp.float32).at[:N_NODE].set(x)

    cnt_parts = _sc_count(dst)
    dinv_b, recip_b = _tc_deg(cnt_parts)

    xs = _tc_mm_scale(xp, W1.T, dinv_b)
    p1 = _sc_gather_scatter(src, dst, xs)
    h = _tc_gcn_combine(p1, xs, dinv_b, b1.reshape(1, HID))

    hs = None
    for Wf, bf, Ws, bs in ((Wf1, bf1, Ws1, bs1), (Wf2, bf2, Ws2, bs2)):
        wfdT = Wf[:, :HID].T
        wsdT = Ws[:, :HID].T
        wfsT = Wf[:, HID:2 * HID].T
        wssT = Ws[:, HID:2 * HID].T
        weT = jnp.concatenate([Wf[:, 2 * HID:], Ws[:, 2 * HID:]], axis=0).T
        dt, st = _tc_tabs(h, wfdT, wsdT, wfsT, wssT,
                          bf.reshape(1, HID), bs.reshape(1, HID))
        gd = _sc_gather(dst, dt)
        gs = _sc_gather(src, st)
        m = _tc_edge(gd, gs, edge_attr, weT)
        pm = _sc_scatter(dst, m)
        h, hs = _tc_cg_combine(pm, h, recip_b, dinv_b)

    p2 = _sc_gather_scatter(src, dst, hs)
    w2T = jnp.zeros((HID, HID), jnp.float32).at[:, :2].set(W2.T)
    b2p = jnp.zeros((1, HID), jnp.float32).at[0, :2].set(b2)
    out = _tc_final(p2, hs, dinv_b, w2T, b2p)
    return out[:N_NODE, :2]
